# Initial kernel scaffold; baseline (speedup 1.0000x reference)
import functools

import jax
import jax.numpy as jnp
from jax.experimental import pallas as pl
from jax.experimental.pallas import tpu as pltpu
from jax.experimental.pallas import tpu_sc as plsc

_K = 8192
_D = 32
_N = 8192
_BN = 512
_NBLK = _N // _BN

_GW = 128


def _argmin_body(x_ref, cb_ref, idx_ref, c2_ref):
    i = pl.program_id(0)
    cb = cb_ref[...]

    @pl.when(i == 0)
    def _():
        c2_col = jnp.sum(cb * cb, axis=1)
        c2_ref[...] = c2_col.reshape(1, _K)

    x = x_ref[...]
    xc = jax.lax.dot_general(
        x, cb, dimension_numbers=(((1,), (1,)), ((), ())),
        preferred_element_type=jnp.float32)
    x2 = jnp.sum(x * x, axis=1, keepdims=True)
    d2 = (x2 - 2.0 * xc) + c2_ref[...]
    dist = jnp.sqrt(jnp.maximum(d2, 0.0))
    minval = jnp.min(dist, axis=1, keepdims=True)
    iota = jax.lax.broadcasted_iota(jnp.int32, (_BN, _K), 1)
    idx = jnp.min(jnp.where(dist == minval, iota, _K), axis=1, keepdims=True)
    idx_ref[...] = idx[None]


def _tc_argmin(x_flat, codebook):
    return pl.pallas_call(
        _argmin_body,
        grid=(_NBLK,),
        in_specs=[
            pl.BlockSpec((_BN, _D), lambda i: (i, 0)),
            pl.BlockSpec((_K, _D), lambda i: (0, 0)),
        ],
        out_specs=pl.BlockSpec((1, _BN, 1), lambda i: (i, 0, 0)),
        out_shape=jax.ShapeDtypeStruct((_NBLK, _BN, 1), jnp.int32),
        scratch_shapes=[pltpu.VMEM((1, _K), jnp.float32)],
    )(x_flat, codebook)


def _sc_gather(codebook, idx_row):
    mesh = plsc.VectorSubcoreMesh(core_axis_name="core", subcore_axis_name="subcore")

    @functools.partial(
        pl.kernel,
        out_type=jax.ShapeDtypeStruct((_N, _D), codebook.dtype),
        mesh=mesh,
    )
    def gather_kernel(cb_hbm, i_hbm, o_hbm):
        def body(i_vmem, o_vmem):
            pltpu.sync_copy(cb_hbm.at[i_vmem.at[0]], o_vmem)

        pltpu.emit_pipeline(
            body,
            grid=(_N // _GW,),
            in_specs=[pl.BlockSpec((1, _GW), index_map=lambda i: (0, i))],
            out_specs=[pl.BlockSpec((_GW, _D), index_map=lambda i: (i, 0))],
            core_axis_name=("core", "subcore"),
            dimension_semantics=(pltpu.PARALLEL,),
        )(i_hbm, o_hbm)

    return gather_kernel(codebook, idx_row)


def kernel(inputs, codebook):
    b, h, w, d = inputs.shape
    x_flat = inputs.reshape(-1, d)
    idx = _tc_argmin(x_flat, codebook)
    idx_row = idx.reshape(1, _N)
    z_q = _sc_gather(codebook, idx_row)
    return z_q.reshape(b, h, w, d)


# speedup vs baseline: 1.0883x; 1.0883x over previous
import functools

import jax
import jax.numpy as jnp
from jax.experimental import pallas as pl
from jax.experimental.pallas import tpu as pltpu
from jax.experimental.pallas import tpu_sc as plsc

_K = 8192
_D = 32
_N = 8192
_BN = 512
_NBLK = _N // _BN

_GW = 128


def _argmin_body(x_ref, cb_ref, idx_ref, c2_ref):
    i = pl.program_id(0)
    cb = cb_ref[...]

    @pl.when(i == 0)
    def _():
        c2_col = jnp.sum(cb * cb, axis=1)
        c2_ref[...] = c2_col.reshape(1, _K)

    x = x_ref[...]
    xc = jax.lax.dot_general(
        x, cb, dimension_numbers=(((1,), (1,)), ((), ())),
        preferred_element_type=jnp.float32)
    x2 = jnp.sum(x * x, axis=1, keepdims=True)
    d2 = (x2 - 2.0 * xc) + c2_ref[...]
    dist = jnp.sqrt(jnp.maximum(d2, 0.0))
    minval = jnp.min(dist, axis=1, keepdims=True)
    iota = jax.lax.broadcasted_iota(jnp.int32, (_BN, _K), 1)
    idx = jnp.min(jnp.where(dist == minval, iota, _K), axis=1, keepdims=True)
    idx_ref[...] = idx[None]


def _tc_argmin(x_flat, codebook):
    return pl.pallas_call(
        _argmin_body,
        grid=(_NBLK,),
        in_specs=[
            pl.BlockSpec((_BN, _D), lambda i: (i, 0)),
            pl.BlockSpec((_K, _D), lambda i: (0, 0)),
        ],
        out_specs=pl.BlockSpec((1, _BN, 1), lambda i: (i, 0, 0)),
        out_shape=jax.ShapeDtypeStruct((_NBLK, _BN, 1), jnp.int32),
        scratch_shapes=[pltpu.VMEM((1, _K), jnp.float32)],
    )(x_flat, codebook)


def _sc_gather(cb_padded, idx_row):
    mesh = plsc.VectorSubcoreMesh(core_axis_name="core", subcore_axis_name="subcore")

    @functools.partial(
        pl.kernel,
        out_type=jax.ShapeDtypeStruct((_N, 128), cb_padded.dtype),
        mesh=mesh,
    )
    def gather_kernel(cb_hbm, i_hbm, o_hbm):
        def body(i_vmem, o_vmem):
            pltpu.sync_copy(cb_hbm.at[i_vmem.at[0]], o_vmem)

        pltpu.emit_pipeline(
            body,
            grid=(_N // _GW,),
            in_specs=[pl.BlockSpec((1, _GW), index_map=lambda i: (0, i))],
            out_specs=[pl.BlockSpec((_GW, 128), index_map=lambda i: (i, 0))],
            core_axis_name=("core", "subcore"),
            dimension_semantics=(pltpu.PARALLEL,),
        )(i_hbm, o_hbm)

    return gather_kernel(cb_padded, idx_row)


def kernel(inputs, codebook):
    b, h, w, d = inputs.shape
    x_flat = inputs.reshape(-1, d)
    idx = _tc_argmin(x_flat, codebook)
    idx_row = idx.reshape(1, _N)
    cb_padded = jnp.pad(codebook, ((0, 0), (0, 128 - _D)))
    z_q = _sc_gather(cb_padded, idx_row)
    return z_q[:, :_D].reshape(b, h, w, d)


# speedup vs baseline: 1.1439x; 1.0512x over previous
import functools

import jax
import jax.numpy as jnp
from jax.experimental import pallas as pl
from jax.experimental.pallas import tpu as pltpu
from jax.experimental.pallas import tpu_sc as plsc

_K = 8192
_D = 32
_N = 8192
_BN = 512
_NBLK = _N // _BN

_GW = 128


def _argmin_body(x_ref, cbt_ref, idx_ref):
    cbt = cbt_ref[...]
    c2 = jnp.sum(cbt * cbt, axis=0, keepdims=True)
    x = x_ref[...]
    xc = jax.lax.dot_general(
        x, cbt, dimension_numbers=(((1,), (0,)), ((), ())),
        preferred_element_type=jnp.float32)
    x2 = jnp.sum(x * x, axis=1, keepdims=True)
    d2 = (x2 - 2.0 * xc) + c2
    dist = jnp.sqrt(jnp.maximum(d2, 0.0))
    minval = jnp.min(dist, axis=1, keepdims=True)
    iota = jax.lax.broadcasted_iota(jnp.int32, (_BN, _K), 1)
    idx = jnp.min(jnp.where(dist == minval, iota, _K), axis=1, keepdims=True)
    idx_ref[...] = idx[None]


def _tc_argmin(x_flat, cbt):
    return pl.pallas_call(
        _argmin_body,
        grid=(_NBLK,),
        in_specs=[
            pl.BlockSpec((_BN, _D), lambda i: (i, 0)),
            pl.BlockSpec((_D, _K), lambda i: (0, 0)),
        ],
        out_specs=pl.BlockSpec((1, _BN, 1), lambda i: (i, 0, 0)),
        out_shape=jax.ShapeDtypeStruct((_NBLK, _BN, 1), jnp.int32),
        compiler_params=pltpu.CompilerParams(
            dimension_semantics=("parallel",),
        ),
    )(x_flat, cbt)


def _sc_gather(cb_padded, idx_row):
    mesh = plsc.VectorSubcoreMesh(core_axis_name="core", subcore_axis_name="subcore")

    @functools.partial(
        pl.kernel,
        out_type=jax.ShapeDtypeStruct((_N, 128), cb_padded.dtype),
        mesh=mesh,
    )
    def gather_kernel(cb_hbm, i_hbm, o_hbm):
        def body(i_vmem, o_vmem):
            pltpu.sync_copy(cb_hbm.at[i_vmem.at[0]], o_vmem)

        pltpu.emit_pipeline(
            body,
            grid=(_N // _GW,),
            in_specs=[pl.BlockSpec((1, _GW), index_map=lambda i: (0, i))],
            out_specs=[pl.BlockSpec((_GW, 128), index_map=lambda i: (i, 0))],
            core_axis_name=("core", "subcore"),
            dimension_semantics=(pltpu.PARALLEL,),
        )(i_hbm, o_hbm)

    return gather_kernel(cb_padded, idx_row)


def kernel(inputs, codebook):
    b, h, w, d = inputs.shape
    x_flat = inputs.reshape(-1, d)
    idx = _tc_argmin(x_flat, codebook.T)
    idx_row = idx.reshape(1, _N)
    cb_padded = jnp.pad(codebook, ((0, 0), (0, 128 - _D)))
    z_q = _sc_gather(cb_padded, idx_row)
    return z_q[:, :_D].reshape(b, h, w, d)


# speedup vs baseline: 1.4522x; 1.2695x over previous
import functools

import jax
import jax.numpy as jnp
from jax.experimental import pallas as pl
from jax.experimental.pallas import tpu as pltpu
from jax.experimental.pallas import tpu_sc as plsc

_K = 8192
_D = 32
_N = 8192
_BN = 512
_NBLK = _N // _BN

_GW = 128


def _argmin_body(x_ref, cbt_ref, idx_ref):
    cbt = cbt_ref[...]
    c2 = jnp.sum(cbt * cbt, axis=0, keepdims=True)
    x = x_ref[...]
    mm = jax.lax.dot_general(
        -2.0 * x, cbt, dimension_numbers=(((1,), (0,)), ((), ())),
        preferred_element_type=jnp.float32)
    x2 = jnp.sum(x * x, axis=1, keepdims=True)
    d2c = jnp.maximum((x2 + mm) + c2, 0.0)
    dist = jnp.where(d2c == 0.0, 0.0, d2c * jax.lax.rsqrt(d2c))
    minval = jnp.min(dist, axis=1, keepdims=True)
    iota = jax.lax.broadcasted_iota(jnp.int32, (_BN, _K), 1)
    idx = jnp.min(jnp.where(dist == minval, iota, _K), axis=1, keepdims=True)
    idx_ref[...] = idx[None]


def _tc_argmin(x_flat, cbt):
    return pl.pallas_call(
        _argmin_body,
        grid=(_NBLK,),
        in_specs=[
            pl.BlockSpec((_BN, _D), lambda i: (i, 0)),
            pl.BlockSpec((_D, _K), lambda i: (0, 0)),
        ],
        out_specs=pl.BlockSpec((1, _BN, 1), lambda i: (i, 0, 0)),
        out_shape=jax.ShapeDtypeStruct((_NBLK, _BN, 1), jnp.int32),
        compiler_params=pltpu.CompilerParams(
            dimension_semantics=("parallel",),
        ),
    )(x_flat, cbt)


def _sc_gather(cb_padded, idx_row):
    mesh = plsc.VectorSubcoreMesh(core_axis_name="core", subcore_axis_name="subcore")

    @functools.partial(
        pl.kernel,
        out_type=jax.ShapeDtypeStruct((_N, 128), cb_padded.dtype),
        mesh=mesh,
    )
    def gather_kernel(cb_hbm, i_hbm, o_hbm):
        def body(i_vmem, o_vmem):
            pltpu.sync_copy(cb_hbm.at[i_vmem.at[0]], o_vmem)

        pltpu.emit_pipeline(
            body,
            grid=(_N // _GW,),
            in_specs=[pl.BlockSpec((1, _GW), index_map=lambda i: (0, i))],
            out_specs=[pl.BlockSpec((_GW, 128), index_map=lambda i: (i, 0))],
            core_axis_name=("core", "subcore"),
            dimension_semantics=(pltpu.PARALLEL,),
        )(i_hbm, o_hbm)

    return gather_kernel(cb_padded, idx_row)


def kernel(inputs, codebook):
    b, h, w, d = inputs.shape
    x_flat = inputs.reshape(-1, d)
    idx = _tc_argmin(x_flat, codebook.T)
    idx_row = idx.reshape(1, _N)
    cb_padded = jnp.pad(codebook, ((0, 0), (0, 128 - _D)))
    z_q = _sc_gather(cb_padded, idx_row)
    return z_q[:, :_D].reshape(b, h, w, d)
